# Initial kernel scaffold; baseline (speedup 1.0000x reference)
import jax
import jax.numpy as jnp
from jax.experimental import pallas as pl


def kernel(img_feat0, img_feat1, img_feat2, img_feat3, verts):
    raise NotImplementedError("write your pallas kernel here")


# speedup vs baseline: 58.3490x; 58.3490x over previous
import functools

import jax
import jax.numpy as jnp
from jax.experimental import pallas as pl
from jax.experimental.pallas import tpu as pltpu

_NV = 100000
_BN = 2000
_LEVELS = ((56, 64), (28, 128), (14, 256), (7, 512))
_HEAD = 64


def _hw_from_verts(v):
    h = 248.0 * (v[:, 1:2] / v[:, 2:3]) + 111.5
    w = 248.0 * (v[:, 0:1] / (-v[:, 2:3])) + 111.5
    return jnp.clip(h, 0.0, 223.0), jnp.clip(w, 0.0, 223.0)


def _body(vh_ref, f0_ref, f1_ref, f2_ref, f3_ref, verts_ref, out_ref):
    verts = verts_ref[...]
    vh = vh_ref[...]
    h, w = _hw_from_verts(verts)
    hh, _ = _hw_from_verts(vh)

    out_ref[:, 0:3] = verts
    col = 3
    jj = jax.lax.broadcasted_iota(jnp.int32, (_BN, _HEAD), 1)
    for f_ref, (s, chan) in zip((f0_ref, f1_ref, f2_ref, f3_ref), _LEVELS):
        inv = 224.0 / s
        x = h / inv
        y = w / inv
        x1 = jnp.floor(x)
        x2 = jnp.ceil(x)
        y1 = jnp.floor(y)
        y2 = jnp.ceil(y)
        gx = x2 - x
        fx = x - x1
        gy = y2 - y
        fy = y - y1
        sel1 = (jj == y1.astype(jnp.int32)).astype(jnp.float32)
        sel2 = (jj == y2.astype(jnp.int32)).astype(jnp.float32)
        smat = gy * sel1 + fy * sel2
        p2 = jnp.concatenate([gx * smat, fx * smat], axis=1)

        xh = hh / inv
        a = jnp.clip(jnp.floor(xh), 0.0, s - 1.0)
        b = jnp.clip(jnp.ceil(xh), 0.0, s - 1.0)
        rcols = jax.lax.broadcasted_iota(jnp.int32, (_HEAD, s), 1)
        a1h = (rcols == a.astype(jnp.int32)).astype(jnp.float32)
        b1h = (rcols == b.astype(jnp.int32)).astype(jnp.float32)
        ab = jnp.concatenate([a1h, b1h], axis=0)

        wmat = jax.lax.dot_general(p2, ab, (((1,), (0,)), ((), ())),
                                   preferred_element_type=jnp.float32)
        lev = jax.lax.dot_general(wmat, f_ref[...], (((1,), (0,)), ((), ())),
                                  preferred_element_type=jnp.float32)
        out_ref[:, col:col + chan] = lev
        col += chan


@jax.jit
def kernel(img_feat0, img_feat1, img_feat2, img_feat3, verts):
    total_c = 3 + sum(c for _, c in _LEVELS)
    grid = _NV // _BN
    const = lambda i: (0, 0)
    return pl.pallas_call(
        _body,
        grid=(grid,),
        in_specs=[
            pl.BlockSpec((_HEAD, 3), const),
            pl.BlockSpec((56, 64), const),
            pl.BlockSpec((28, 128), const),
            pl.BlockSpec((14, 256), const),
            pl.BlockSpec((7, 512), const),
            pl.BlockSpec((_BN, 3), lambda i: (i, 0)),
        ],
        out_specs=pl.BlockSpec((_BN, total_c), lambda i: (i, 0)),
        out_shape=jax.ShapeDtypeStruct((_NV, total_c), jnp.float32),
        compiler_params=pltpu.CompilerParams(
            dimension_semantics=("arbitrary",),
        ),
    )(verts, img_feat0, img_feat1, img_feat2, img_feat3, verts)


# speedup vs baseline: 65.0761x; 1.1153x over previous
import numpy as np

import jax
import jax.numpy as jnp
from jax.experimental import pallas as pl
from jax.experimental.pallas import tpu as pltpu

_NV = 100000
_BN = 2000
_LEVELS = ((56, 64), (28, 128), (14, 256), (7, 512))
_HEAD = 64
_STRIP = 8 * _HEAD

_HIGH = jax.lax.Precision.HIGHEST


def _strip_consts():
    scale = np.zeros((_STRIP,), np.float32)
    jloc = np.zeros((_STRIP,), np.float32)
    e8 = np.zeros((8, _STRIP), np.float32)
    for lev, (s, _) in enumerate(_LEVELS):
        for k in range(2 * _HEAD):
            lane = 2 * _HEAD * lev + k
            scale[lane] = s / 224.0
            jloc[lane] = k % _HEAD
            e8[2 * lev + k // _HEAD, lane] = 1.0
    k1 = np.stack([scale, -jloc])
    return jnp.asarray(k1), jnp.asarray(e8)


def _hw_from_verts(v):
    h = 248.0 * (v[:, 1:2] / v[:, 2:3]) + 111.5
    w = 248.0 * (v[:, 0:1] / (-v[:, 2:3])) + 111.5
    return jnp.clip(h, 0.0, 223.0), jnp.clip(w, 0.0, 223.0)


def _mm(a, b, precision=None):
    return jax.lax.dot_general(a, b, (((1,), (0,)), ((), ())),
                               preferred_element_type=jnp.float32,
                               precision=precision)


def _body(vh_ref, f0_ref, f1_ref, f2_ref, f3_ref, k1_ref, e8_ref,
          verts_ref, out_ref):
    verts = verts_ref[...]
    vh = vh_ref[...]
    h, w = _hw_from_verts(verts)
    hh, _ = _hw_from_verts(vh)

    win = jnp.concatenate([w, jnp.ones_like(w)], axis=1)
    d = _mm(win, k1_ref[...], _HIGH)
    tent = jnp.maximum(1.0 - jnp.abs(d), 0.0)

    cols = []
    for s, _ in _LEVELS:
        inv = 224.0 / s
        x = h / inv
        y = w / inv
        nz = (jnp.floor(y) != jnp.ceil(y)).astype(jnp.float32)
        cols.append((jnp.ceil(x) - x) * nz)
        cols.append((x - jnp.floor(x)) * nz)
    f8 = jnp.concatenate(cols, axis=1)
    gfac = _mm(f8, e8_ref[...], _HIGH)
    w2 = gfac * tent

    out_ref[:, 0:3] = verts
    col = 3
    rcols = jax.lax.broadcasted_iota(jnp.int32, (_HEAD, 1), 0)
    for lev, (f_ref, (s, chan)) in enumerate(
            zip((f0_ref, f1_ref, f2_ref, f3_ref), _LEVELS)):
        inv = 224.0 / s
        xh = hh / inv
        a = jnp.clip(jnp.floor(xh), 0.0, s - 1.0).astype(jnp.int32)
        b = jnp.clip(jnp.ceil(xh), 0.0, s - 1.0).astype(jnp.int32)
        rr = jax.lax.broadcasted_iota(jnp.int32, (_HEAD, s), 1)
        a1h = (rr == a).astype(jnp.float32)
        b1h = (rr == b).astype(jnp.float32)
        ab = jnp.concatenate([a1h, b1h], axis=0)
        tab = _mm(ab, f_ref[...], _HIGH)
        w2l = jax.lax.slice_in_dim(w2, 2 * _HEAD * lev,
                                   2 * _HEAD * (lev + 1), axis=1)
        out_ref[:, col:col + chan] = _mm(w2l, tab)
        col += chan


@jax.jit
def kernel(img_feat0, img_feat1, img_feat2, img_feat3, verts):
    total_c = 3 + sum(c for _, c in _LEVELS)
    grid = _NV // _BN
    k1, e8 = _strip_consts()
    const = lambda i: (0, 0)
    return pl.pallas_call(
        _body,
        grid=(grid,),
        in_specs=[
            pl.BlockSpec((_HEAD, 3), const),
            pl.BlockSpec((56, 64), const),
            pl.BlockSpec((28, 128), const),
            pl.BlockSpec((14, 256), const),
            pl.BlockSpec((7, 512), const),
            pl.BlockSpec((2, _STRIP), const),
            pl.BlockSpec((8, _STRIP), const),
            pl.BlockSpec((_BN, 3), lambda i: (i, 0)),
        ],
        out_specs=pl.BlockSpec((_BN, total_c), lambda i: (i, 0)),
        out_shape=jax.ShapeDtypeStruct((_NV, total_c), jnp.float32),
        compiler_params=pltpu.CompilerParams(
            dimension_semantics=("arbitrary",),
        ),
    )(verts, img_feat0, img_feat1, img_feat2, img_feat3, k1, e8, verts)


# speedup vs baseline: 75.2443x; 1.1563x over previous
import numpy as np

import jax
import jax.numpy as jnp
from jax.experimental import pallas as pl
from jax.experimental.pallas import tpu as pltpu

_NV = 100000
_BN = 2000
_LEVELS = ((56, 64), (28, 128), (14, 256), (7, 512))
_BW = (64, 32, 16, 8)
_STRIP = 2 * sum(_BW)
_SPAD = 256


def _strip_consts():
    scale = np.zeros((_SPAD,), np.float32)
    jloc = np.full((_SPAD,), 1.0e6, np.float32)
    e8 = np.zeros((8, _SPAD), np.float32)
    lane = 0
    for lev, (s, _) in enumerate(_LEVELS):
        bw = _BW[lev]
        for half in range(2):
            for j in range(bw):
                scale[lane] = s / 224.0
                jloc[lane] = j
                e8[2 * lev + half, lane] = 1.0
                lane += 1
    k1 = np.stack([scale, -jloc])
    return jnp.asarray(k1), jnp.asarray(e8)


def _hw_from_verts(v):
    h = 248.0 * (v[:, 1:2] / v[:, 2:3]) + 111.5
    w = 248.0 * (v[:, 0:1] / (-v[:, 2:3])) + 111.5
    return jnp.clip(h, 0.0, 223.0), jnp.clip(w, 0.0, 223.0)


def _mm(a, b, precision=None):
    return jax.lax.dot_general(a, b, (((1,), (0,)), ((), ())),
                               preferred_element_type=jnp.float32,
                               precision=precision)


def _body(vh_ref, f0_ref, f1_ref, f2_ref, f3_ref, k1_ref, e8_ref,
          verts_ref, out_ref):
    verts = verts_ref[...]
    vh = vh_ref[...]
    h, w = _hw_from_verts(verts)
    hh, _ = _hw_from_verts(vh)

    win = jnp.concatenate([w, jnp.ones_like(w)], axis=1)
    d = _mm(win, k1_ref[...], jax.lax.Precision.HIGHEST)
    tent = jnp.maximum(1.0 - jnp.abs(d), 0.0)

    cols = []
    for s, _ in _LEVELS:
        inv = 224.0 / s
        x = h / inv
        y = w / inv
        nz = (jnp.floor(y) != jnp.ceil(y)).astype(jnp.float32)
        cols.append((jnp.ceil(x) - x) * nz)
        cols.append((x - jnp.floor(x)) * nz)
    f8 = jnp.concatenate(cols, axis=1)
    gfac = _mm(f8, e8_ref[...])
    w2 = gfac * tent

    out_ref[:, 0:3] = verts
    col = 3
    off = 0
    for lev, (f_ref, (s, chan)) in enumerate(
            zip((f0_ref, f1_ref, f2_ref, f3_ref), _LEVELS)):
        bw = _BW[lev]
        inv = 224.0 / s
        xh = hh / inv
        a = jnp.clip(jnp.floor(xh), 0.0, s - 1.0).astype(jnp.int32)
        b = jnp.clip(jnp.ceil(xh), 0.0, s - 1.0).astype(jnp.int32)
        ah = jax.lax.slice_in_dim(a, 0, bw, axis=0)
        bh = jax.lax.slice_in_dim(b, 0, bw, axis=0)
        rr = jax.lax.broadcasted_iota(jnp.int32, (bw, s), 1)
        a1h = (rr == ah).astype(jnp.float32)
        b1h = (rr == bh).astype(jnp.float32)
        ab = jnp.concatenate([a1h, b1h], axis=0)
        tab = _mm(ab, f_ref[...], jax.lax.Precision.HIGHEST)
        w2l = jax.lax.slice_in_dim(w2, off, off + 2 * bw, axis=1)
        out_ref[:, col:col + chan] = _mm(w2l, tab)
        col += chan
        off += 2 * bw


@jax.jit
def kernel(img_feat0, img_feat1, img_feat2, img_feat3, verts):
    total_c = 3 + sum(c for _, c in _LEVELS)
    grid = _NV // _BN
    k1, e8 = _strip_consts()
    const = lambda i: (0, 0)
    return pl.pallas_call(
        _body,
        grid=(grid,),
        in_specs=[
            pl.BlockSpec((64, 3), const),
            pl.BlockSpec((56, 64), const),
            pl.BlockSpec((28, 128), const),
            pl.BlockSpec((14, 256), const),
            pl.BlockSpec((7, 512), const),
            pl.BlockSpec((2, _SPAD), const),
            pl.BlockSpec((8, _SPAD), const),
            pl.BlockSpec((_BN, 3), lambda i: (i, 0)),
        ],
        out_specs=pl.BlockSpec((_BN, total_c), lambda i: (i, 0)),
        out_shape=jax.ShapeDtypeStruct((_NV, total_c), jnp.float32),
        compiler_params=pltpu.CompilerParams(
            dimension_semantics=("arbitrary",),
        ),
    )(verts, img_feat0, img_feat1, img_feat2, img_feat3, k1, e8, verts)
